# Initial kernel scaffold; baseline (speedup 1.0000x reference)
import jax
import jax.numpy as jnp
from jax.experimental import pallas as pl


def kernel(x, edge_index, W1, b1, W2, b2):
    raise NotImplementedError("write your pallas kernel here")


# speedup vs baseline: 27.6207x; 27.6207x over previous
import functools

import jax
import jax.numpy as jnp
from jax import lax
from jax.experimental import pallas as pl
from jax.experimental.pallas import tpu as pltpu
from jax.experimental.pallas import tpu_sc as plsc

NC = 2
NS = 16
NW = NC * NS
CHUNK = 125
DEGW = 16

_f32 = jnp.float32


def _sc_mesh():
    return plsc.VectorSubcoreMesh(
        core_axis_name="c", subcore_axis_name="s", num_cores=NC, num_subcores=NS
    )


def _make_deg_kernel(n_nodes: int, n_chunks: int):
    rpt = n_nodes // NS

    @functools.partial(
        pl.kernel,
        out_type=jax.ShapeDtypeStruct((NC, n_nodes, DEGW), _f32),
        mesh=_sc_mesh(),
        compiler_params=pltpu.CompilerParams(use_tc_tiling_on_sc=False),
        scratch_types=[
            pltpu.VMEM((n_chunks, CHUNK), jnp.int32),
            pltpu.VMEM((CHUNK, DEGW), _f32),
            pltpu.VMEM_SHARED((n_nodes, DEGW), _f32),
        ],
    )
    def deg_kernel(dst_hbm, ones_hbm, zeros_hbm, out_hbm, dst_v, ones_v, acc):
        c = lax.axis_index("c")
        s = lax.axis_index("s")
        pltpu.sync_copy(dst_hbm.at[c, s], dst_v)
        pltpu.sync_copy(ones_hbm, ones_v)
        pltpu.sync_copy(zeros_hbm, acc.at[pl.ds(s * rpt, rpt)])
        plsc.subcore_barrier()

        def body(j, carry):
            pltpu.sync_copy(ones_v, acc.at[dst_v.at[j]], add=True)
            return carry

        lax.fori_loop(0, n_chunks, body, 0)
        plsc.subcore_barrier()
        pltpu.sync_copy(
            acc.at[pl.ds(s * rpt, rpt)], out_hbm.at[c, pl.ds(s * rpt, rpt)]
        )

    return deg_kernel


IDXG = 8


def _make_seg_sum_kernel(n_nodes: int, n_chunks: int, d: int):
    rpt = n_nodes // NS
    assert n_chunks % IDXG == 0
    n_groups = n_chunks // IDXG
    params = None if d % 128 == 0 else pltpu.CompilerParams(use_tc_tiling_on_sc=False)

    @functools.partial(
        pl.kernel,
        out_type=jax.ShapeDtypeStruct((NC, n_nodes, d), _f32),
        mesh=_sc_mesh(),
        compiler_params=params,
        scratch_types=[
            pltpu.VMEM((IDXG, CHUNK), jnp.int32),
            pltpu.VMEM((IDXG, CHUNK), jnp.int32),
            pltpu.VMEM((CHUNK, d), _f32),
            pltpu.VMEM((CHUNK, d), _f32),
            pltpu.VMEM_SHARED((n_nodes, d), _f32),
            pltpu.SemaphoreType.DMA,
            pltpu.SemaphoreType.DMA,
        ],
    )
    def seg_sum(table_hbm, src_hbm, dst_hbm, zeros_hbm, out_hbm,
                src_v, dst_v, buf0, buf1, acc, sem0, sem1):
        c = lax.axis_index("c")
        s = lax.axis_index("s")
        pltpu.sync_copy(zeros_hbm, acc.at[pl.ds(s * rpt, rpt)])
        plsc.subcore_barrier()

        bufs = (buf0, buf1)
        sems = (sem0, sem1)

        def group(g, carry):
            pltpu.sync_copy(src_hbm.at[c, s, pl.ds(g * IDXG, IDXG)], src_v)
            pltpu.sync_copy(dst_hbm.at[c, s, pl.ds(g * IDXG, IDXG)], dst_v)
            pltpu.async_copy(table_hbm.at[src_v.at[0]], buf0, sem0)
            for b in range(IDXG):
                bb = b % 2
                pltpu.make_async_copy(
                    table_hbm.at[src_v.at[b]], bufs[bb], sems[bb]
                ).wait()
                if b + 1 < IDXG:
                    nb = (b + 1) % 2
                    pltpu.async_copy(
                        table_hbm.at[src_v.at[b + 1]], bufs[nb], sems[nb]
                    )
                pltpu.sync_copy(bufs[bb], acc.at[dst_v.at[b]], add=True)
            return carry

        lax.fori_loop(0, n_groups, group, 0)
        plsc.subcore_barrier()
        pltpu.sync_copy(
            acc.at[pl.ds(s * rpt, rpt)], out_hbm.at[c, pl.ds(s * rpt, rpt)]
        )

    return seg_sum


def _tc_stage_a(x, w1, degp, bn: int):
    n, d_in = x.shape
    d_h = w1.shape[1]

    def body(x_ref, w1_ref, degp_ref, table_ref, dinv_ref):
        deg = (
            jnp.sum(degp_ref[0], axis=-1) + jnp.sum(degp_ref[1], axis=-1)
        ) * (1.0 / DEGW) + 1.0
        dinv = lax.rsqrt(deg)
        p = jnp.dot(x_ref[...], w1_ref[...], preferred_element_type=_f32)
        table_ref[...] = p * dinv[:, None]
        dinv_ref[...] = dinv[:, None]

    return pl.pallas_call(
        body,
        grid=(n // bn,),
        in_specs=[
            pl.BlockSpec((bn, d_in), lambda i: (i, 0)),
            pl.BlockSpec((d_in, d_h), lambda i: (0, 0)),
            pl.BlockSpec((NC, bn, DEGW), lambda i: (0, i, 0)),
        ],
        out_specs=[
            pl.BlockSpec((bn, d_h), lambda i: (i, 0)),
            pl.BlockSpec((bn, 1), lambda i: (i, 0)),
        ],
        out_shape=[
            jax.ShapeDtypeStruct((n, d_h), _f32),
            jax.ShapeDtypeStruct((n, 1), _f32),
        ],
    )(x, w1, degp)


def _tc_stage_b(aggp, table1, dinv, b1, w2p, bn: int):
    n, d_h = table1.shape
    d2 = w2p.shape[1]

    def body(aggp_ref, t1_ref, dinv_ref, b1_ref, w2_ref, out_ref):
        dv = dinv_ref[...]
        h = dv * (aggp_ref[0] + aggp_ref[1] + t1_ref[...]) + b1_ref[...]
        h = jnp.maximum(h, 0.0)
        q = jnp.dot(h, w2_ref[...], preferred_element_type=_f32)
        out_ref[...] = q * dv

    return pl.pallas_call(
        body,
        grid=(n // bn,),
        in_specs=[
            pl.BlockSpec((NC, bn, d_h), lambda i: (0, i, 0)),
            pl.BlockSpec((bn, d_h), lambda i: (i, 0)),
            pl.BlockSpec((bn, 1), lambda i: (i, 0)),
            pl.BlockSpec((1, d_h), lambda i: (0, 0)),
            pl.BlockSpec((d_h, d2), lambda i: (0, 0)),
        ],
        out_specs=pl.BlockSpec((bn, d2), lambda i: (i, 0)),
        out_shape=jax.ShapeDtypeStruct((n, d2), _f32),
    )(aggp, table1, dinv, b1, w2p)


def _tc_stage_c(aggp, table2, dinv, b2p, d_out: int, bn: int):
    n, d2 = table2.shape

    def body(aggp_ref, t2_ref, dinv_ref, b2_ref, out_ref):
        dv = dinv_ref[...]
        o = dv * (aggp_ref[0] + aggp_ref[1] + t2_ref[...]) + b2_ref[...]
        col = lax.broadcasted_iota(jnp.int32, o.shape, 1)
        valid = col < d_out
        neg = jnp.full_like(o, -jnp.inf)
        logits = jnp.where(valid, o, neg)
        m = jnp.max(logits, axis=-1, keepdims=True)
        lse = jnp.log(jnp.sum(jnp.exp(logits - m), axis=-1, keepdims=True)) + m
        out_ref[...] = (o - lse)[:, :d_out]

    return pl.pallas_call(
        body,
        grid=(n // bn,),
        in_specs=[
            pl.BlockSpec((NC, bn, d2), lambda i: (0, i, 0)),
            pl.BlockSpec((bn, d2), lambda i: (i, 0)),
            pl.BlockSpec((bn, 1), lambda i: (i, 0)),
            pl.BlockSpec((1, d2), lambda i: (0, 0)),
        ],
        out_specs=pl.BlockSpec((bn, d_out), lambda i: (i, 0)),
        out_shape=jax.ShapeDtypeStruct((n, d_out), _f32),
    )(aggp, table2, dinv, b2p)


def kernel(x, edge_index, W1, b1, W2, b2):
    n, d_in = x.shape
    e = edge_index.shape[1]
    d_h = W1.shape[1]
    d_out = W2.shape[1]
    n_chunks = e // (NW * CHUNK)
    assert e == NW * n_chunks * CHUNK, "edge count must tile over workers"

    d2 = 48
    npad = -(-n // (NS * 8)) * (NS * 8)
    assert (e // (NW * CHUNK)) % IDXG == 0

    src = edge_index[0].reshape(NC, NS, n_chunks, CHUNK)
    dst = edge_index[1].reshape(NC, NS, n_chunks, CHUNK)

    rpt = npad // NS
    ones_deg = jnp.ones((CHUNK, DEGW), _f32)
    zeros_deg = jnp.zeros((rpt, DEGW), _f32)
    zeros_h = jnp.zeros((rpt, d_h), _f32)
    zeros_2 = jnp.zeros((rpt, d2), _f32)
    w2p = jnp.pad(W2, ((0, 0), (0, d2 - d_out)))
    b1r = b1.reshape(1, d_h)
    b2p = jnp.pad(b2, (0, d2 - d_out)).reshape(1, d2)

    degp = _make_deg_kernel(npad, n_chunks)(dst, ones_deg, zeros_deg)
    table1, dinv = _tc_stage_a(x, W1, degp, bn=2000)
    aggp1 = _make_seg_sum_kernel(npad, n_chunks, d_h)(table1, src, dst, zeros_h)
    table2 = _tc_stage_b(aggp1, table1, dinv, b1r, w2p, bn=2000)
    aggp2 = _make_seg_sum_kernel(npad, n_chunks, d2)(table2, src, dst, zeros_2)
    return _tc_stage_c(aggp2, table2, dinv, b2p, d_out, bn=2000)


# speedup vs baseline: 29.6929x; 1.0750x over previous
import functools

import jax
import jax.numpy as jnp
from jax import lax
from jax.experimental import pallas as pl
from jax.experimental.pallas import tpu as pltpu
from jax.experimental.pallas import tpu_sc as plsc

NC = 2
NS = 16
NW = NC * NS
CHUNK = 128
DEGW = 16

_f32 = jnp.float32


def _sc_mesh():
    return plsc.VectorSubcoreMesh(
        core_axis_name="c", subcore_axis_name="s", num_cores=NC, num_subcores=NS
    )


def _make_deg_kernel(n_nodes: int, n_chunks: int):
    rpt = n_nodes // NS

    @functools.partial(
        pl.kernel,
        out_type=jax.ShapeDtypeStruct((NC, n_nodes, DEGW), _f32),
        mesh=_sc_mesh(),
        compiler_params=pltpu.CompilerParams(use_tc_tiling_on_sc=False),
        scratch_types=[
            pltpu.VMEM((n_chunks, 1, CHUNK), jnp.int32),
            pltpu.VMEM((CHUNK, DEGW), _f32),
            pltpu.VMEM_SHARED((n_nodes, DEGW), _f32),
        ],
    )
    def deg_kernel(dst_hbm, ones_hbm, zeros_hbm, out_hbm, dst_v, ones_v, acc):
        c = lax.axis_index("c")
        s = lax.axis_index("s")
        pltpu.sync_copy(dst_hbm.at[c, s], dst_v)
        pltpu.sync_copy(ones_hbm, ones_v)
        pltpu.sync_copy(zeros_hbm, acc.at[pl.ds(s * rpt, rpt)])
        plsc.subcore_barrier()

        def body(j, carry):
            pltpu.sync_copy(ones_v, acc.at[dst_v.at[j, 0]], add=True)
            return carry

        lax.fori_loop(0, n_chunks, body, 0)
        plsc.subcore_barrier()
        pltpu.sync_copy(
            acc.at[pl.ds(s * rpt, rpt)], out_hbm.at[c, pl.ds(s * rpt, rpt)]
        )

    return deg_kernel


def _make_seg_sum_kernel(n_nodes: int, n_chunks: int, d: int):
    rpt = n_nodes // NS
    assert n_chunks % 2 == 0 and n_chunks >= 4
    params = None if d % 128 == 0 else pltpu.CompilerParams(use_tc_tiling_on_sc=False)

    @functools.partial(
        pl.kernel,
        out_type=jax.ShapeDtypeStruct((NC, n_nodes, d), _f32),
        mesh=_sc_mesh(),
        compiler_params=params,
        scratch_types=[
            pltpu.VMEM((CHUNK,), jnp.int32),
            pltpu.VMEM((CHUNK,), jnp.int32),
            pltpu.VMEM((CHUNK,), jnp.int32),
            pltpu.VMEM((CHUNK,), jnp.int32),
            pltpu.VMEM((CHUNK, d), _f32),
            pltpu.VMEM((CHUNK, d), _f32),
            pltpu.VMEM_SHARED((n_nodes, d), _f32),
            pltpu.SemaphoreType.DMA,
            pltpu.SemaphoreType.DMA,
            pltpu.SemaphoreType.DMA,
            pltpu.SemaphoreType.DMA,
            pltpu.SemaphoreType.DMA,
            pltpu.SemaphoreType.DMA,
            pltpu.SemaphoreType.DMA,
            pltpu.SemaphoreType.DMA,
        ],
    )
    def seg_sum(table_hbm, src_hbm, dst_hbm, junk_hbm, zeros_hbm, out_hbm,
                sidx0, sidx1, didx0, didx1, buf0, buf1, acc,
                gsem0, gsem1, ssem0, ssem1, is0, is1, id0, id1):
        c = lax.axis_index("c")
        s = lax.axis_index("s")
        pltpu.sync_copy(zeros_hbm, acc.at[pl.ds(s * rpt, rpt)])
        plsc.subcore_barrier()

        sidx = (sidx0, sidx1)
        didx = (didx0, didx1)
        bufs = (buf0, buf1)
        gsem = (gsem0, gsem1)
        ssem = (ssem0, ssem1)
        isem = (is0, is1)
        dsem = (id0, id1)

        pltpu.async_copy(src_hbm.at[c, s, 0, 0], sidx0, is0)
        pltpu.async_copy(src_hbm.at[c, s, 1, 0], sidx1, is1)
        pltpu.async_copy(dst_hbm.at[c, s, 0, 0], didx0, id0)
        pltpu.sync_copy(junk_hbm, didx1)
        pltpu.make_async_copy(src_hbm.at[c, s, 0, 0], sidx0, is0).wait()
        pltpu.async_copy(table_hbm.at[sidx0], buf0, gsem0)
        pltpu.async_copy(buf1, acc.at[didx1], ssem1, add=True)

        def one(j, b):
            nb = 1 - b
            pltpu.make_async_copy(table_hbm.at[sidx[b]], bufs[b], gsem[b]).wait()
            pltpu.async_copy(src_hbm.at[c, s, j + 2, 0], sidx[b], isem[b])
            pltpu.make_async_copy(bufs[nb], acc.at[didx[nb]], ssem[nb]).wait()
            pltpu.async_copy(dst_hbm.at[c, s, j + 1, 0], didx[nb], dsem[nb])
            pltpu.make_async_copy(src_hbm.at[c, s, j + 1, 0], sidx[nb], isem[nb]).wait()
            pltpu.async_copy(table_hbm.at[sidx[nb]], bufs[nb], gsem[nb])
            pltpu.make_async_copy(dst_hbm.at[c, s, j, 0], didx[b], dsem[b]).wait()
            pltpu.async_copy(bufs[b], acc.at[didx[b]], ssem[b], add=True)

        def pair(i, carry):
            one(2 * i, 0)
            one(2 * i + 1, 1)
            return carry

        lax.fori_loop(0, (n_chunks - 2) // 2, pair, 0)

        jt = n_chunks - 2
        pltpu.make_async_copy(table_hbm.at[sidx0], buf0, gsem0).wait()
        pltpu.make_async_copy(buf1, acc.at[didx1], ssem1).wait()
        pltpu.async_copy(dst_hbm.at[c, s, jt + 1, 0], didx1, id1)
        pltpu.make_async_copy(src_hbm.at[c, s, jt + 1, 0], sidx1, is1).wait()
        pltpu.async_copy(table_hbm.at[sidx1], buf1, gsem1)
        pltpu.make_async_copy(dst_hbm.at[c, s, jt, 0], didx0, id0).wait()
        pltpu.async_copy(buf0, acc.at[didx0], ssem0, add=True)
        pltpu.make_async_copy(table_hbm.at[sidx1], buf1, gsem1).wait()
        pltpu.make_async_copy(buf0, acc.at[didx0], ssem0).wait()
        pltpu.make_async_copy(dst_hbm.at[c, s, jt + 1, 0], didx1, id1).wait()
        pltpu.async_copy(buf1, acc.at[didx1], ssem1, add=True)
        pltpu.make_async_copy(buf1, acc.at[didx1], ssem1).wait()

        plsc.subcore_barrier()
        pltpu.sync_copy(
            acc.at[pl.ds(s * rpt, rpt)], out_hbm.at[c, pl.ds(s * rpt, rpt)]
        )

    return seg_sum


def _tc_stage_a(x, w1, degp, bn: int):
    n, d_in = x.shape
    d_h = w1.shape[1]

    def body(x_ref, w1_ref, degp_ref, table_ref, dinv_ref):
        deg = (
            jnp.sum(degp_ref[0], axis=-1) + jnp.sum(degp_ref[1], axis=-1)
        ) * (1.0 / DEGW) + 1.0
        dinv = lax.rsqrt(deg)
        p = jnp.dot(x_ref[...], w1_ref[...], preferred_element_type=_f32)
        table_ref[...] = p * dinv[:, None]
        dinv_ref[...] = dinv[:, None]

    return pl.pallas_call(
        body,
        grid=(n // bn,),
        in_specs=[
            pl.BlockSpec((bn, d_in), lambda i: (i, 0)),
            pl.BlockSpec((d_in, d_h), lambda i: (0, 0)),
            pl.BlockSpec((NC, bn, DEGW), lambda i: (0, i, 0)),
        ],
        out_specs=[
            pl.BlockSpec((bn, d_h), lambda i: (i, 0)),
            pl.BlockSpec((bn, 1), lambda i: (i, 0)),
        ],
        out_shape=[
            jax.ShapeDtypeStruct((n, d_h), _f32),
            jax.ShapeDtypeStruct((n, 1), _f32),
        ],
    )(x, w1, degp)


def _tc_stage_b(aggp, table1, dinv, b1, w2p, bn: int):
    n, d_h = table1.shape
    d2 = w2p.shape[1]

    def body(aggp_ref, t1_ref, dinv_ref, b1_ref, w2_ref, out_ref):
        dv = dinv_ref[...]
        h = dv * (aggp_ref[0] + aggp_ref[1] + t1_ref[...]) + b1_ref[...]
        h = jnp.maximum(h, 0.0)
        q = jnp.dot(h, w2_ref[...], preferred_element_type=_f32)
        out_ref[...] = q * dv

    return pl.pallas_call(
        body,
        grid=(n // bn,),
        in_specs=[
            pl.BlockSpec((NC, bn, d_h), lambda i: (0, i, 0)),
            pl.BlockSpec((bn, d_h), lambda i: (i, 0)),
            pl.BlockSpec((bn, 1), lambda i: (i, 0)),
            pl.BlockSpec((1, d_h), lambda i: (0, 0)),
            pl.BlockSpec((d_h, d2), lambda i: (0, 0)),
        ],
        out_specs=pl.BlockSpec((bn, d2), lambda i: (i, 0)),
        out_shape=jax.ShapeDtypeStruct((n, d2), _f32),
    )(aggp, table1, dinv, b1, w2p)


def _tc_stage_c(aggp, table2, dinv, b2p, d_out: int, bn: int):
    n, d2 = table2.shape

    def body(aggp_ref, t2_ref, dinv_ref, b2_ref, out_ref):
        dv = dinv_ref[...]
        o = dv * (aggp_ref[0] + aggp_ref[1] + t2_ref[...]) + b2_ref[...]
        col = lax.broadcasted_iota(jnp.int32, o.shape, 1)
        valid = col < d_out
        neg = jnp.full_like(o, -jnp.inf)
        logits = jnp.where(valid, o, neg)
        m = jnp.max(logits, axis=-1, keepdims=True)
        lse = jnp.log(jnp.sum(jnp.exp(logits - m), axis=-1, keepdims=True)) + m
        out_ref[...] = (o - lse)[:, :d_out]

    return pl.pallas_call(
        body,
        grid=(n // bn,),
        in_specs=[
            pl.BlockSpec((NC, bn, d2), lambda i: (0, i, 0)),
            pl.BlockSpec((bn, d2), lambda i: (i, 0)),
            pl.BlockSpec((bn, 1), lambda i: (i, 0)),
            pl.BlockSpec((1, d2), lambda i: (0, 0)),
        ],
        out_specs=pl.BlockSpec((bn, d_out), lambda i: (i, 0)),
        out_shape=jax.ShapeDtypeStruct((n, d_out), _f32),
    )(aggp, table2, dinv, b2p)


def kernel(x, edge_index, W1, b1, W2, b2):
    n, d_in = x.shape
    e = edge_index.shape[1]
    d_h = W1.shape[1]
    d_out = W2.shape[1]
    d2 = 48
    npad = -(-n // (NS * 8)) * (NS * 8) + (NS * 8 if n % (NS * 8) == 0 else 0)

    assert e % NW == 0
    epw = e // NW
    n_chunks = -(-epw // CHUNK)
    n_chunks += n_chunks % 2
    npp = n_chunks * CHUNK - epw

    srcw = edge_index[0].reshape(NW, epw)
    dstw = edge_index[1].reshape(NW, epw)
    if npp:
        wid = jnp.arange(NW, dtype=jnp.int32)[:, None]
        k = jnp.arange(npp, dtype=jnp.int32)[None, :]
        pad_src = (k * 131 + wid * 977) % n
        pad_dst = n + (k + wid * 7) % (npad - n)
        srcw = jnp.concatenate([srcw, pad_src], axis=1)
        dstw = jnp.concatenate([dstw, pad_dst], axis=1)
    src = srcw.reshape(NC, NS, n_chunks, 1, CHUNK)
    dst = dstw.reshape(NC, NS, n_chunks, 1, CHUNK)

    rpt = npad // NS
    ones_deg = jnp.ones((CHUNK, DEGW), _f32)
    zeros_deg = jnp.zeros((rpt, DEGW), _f32)
    zeros_h = jnp.zeros((rpt, d_h), _f32)
    zeros_2 = jnp.zeros((rpt, d2), _f32)
    w2p = jnp.pad(W2, ((0, 0), (0, d2 - d_out)))
    b1r = b1.reshape(1, d_h)
    b2p = jnp.pad(b2, (0, d2 - d_out)).reshape(1, d2)

    junk = (n + jnp.arange(CHUNK, dtype=jnp.int32) % (npad - n)).astype(jnp.int32)

    degp = _make_deg_kernel(npad, n_chunks)(dst, ones_deg, zeros_deg)
    table1, dinv = _tc_stage_a(x, W1, degp, bn=2000)
    aggp1 = _make_seg_sum_kernel(npad, n_chunks, d_h)(table1, src, dst, junk, zeros_h)
    table2 = _tc_stage_b(aggp1, table1, dinv, b1r, w2p, bn=2000)
    aggp2 = _make_seg_sum_kernel(npad, n_chunks, d2)(table2, src, dst, junk, zeros_2)
    return _tc_stage_c(aggp2, table2, dinv, b2p, d_out, bn=2000)


# speedup vs baseline: 38.1705x; 1.2855x over previous
import functools

import jax
import jax.numpy as jnp
from jax import lax
from jax.experimental import pallas as pl
from jax.experimental.pallas import tpu as pltpu
from jax.experimental.pallas import tpu_sc as plsc

NC = 2
NS = 16
NW = NC * NS
CHUNK = 128
DEGW = 16

_f32 = jnp.float32


def _sc_mesh():
    return plsc.VectorSubcoreMesh(
        core_axis_name="c", subcore_axis_name="s", num_cores=NC, num_subcores=NS
    )


def _make_deg_kernel(n_nodes: int, n_chunks: int):
    rpt = n_nodes // NS

    @functools.partial(
        pl.kernel,
        out_type=jax.ShapeDtypeStruct((NC, n_nodes, DEGW), _f32),
        mesh=_sc_mesh(),
        compiler_params=pltpu.CompilerParams(use_tc_tiling_on_sc=False),
        scratch_types=[
            pltpu.VMEM((n_chunks, 1, CHUNK), jnp.int32),
            pltpu.VMEM((CHUNK, DEGW), _f32),
            pltpu.VMEM_SHARED((n_nodes, DEGW), _f32),
        ],
    )
    def deg_kernel(dst_hbm, ones_hbm, zeros_hbm, out_hbm, dst_v, ones_v, acc):
        c = lax.axis_index("c")
        s = lax.axis_index("s")
        pltpu.sync_copy(dst_hbm.at[c, s], dst_v)
        pltpu.sync_copy(ones_hbm, ones_v)
        pltpu.sync_copy(zeros_hbm, acc.at[pl.ds(s * rpt, rpt)])
        plsc.subcore_barrier()

        def body(j, carry):
            pltpu.sync_copy(ones_v, acc.at[dst_v.at[j, 0]], add=True)
            return carry

        lax.fori_loop(0, n_chunks, body, 0)
        plsc.subcore_barrier()
        pltpu.sync_copy(
            acc.at[pl.ds(s * rpt, rpt)], out_hbm.at[c, pl.ds(s * rpt, rpt)]
        )

    return deg_kernel


def _make_seg_sum_kernel(n_nodes: int, n_chunks: int, d: int, chunk: int, nbuf: int):
    rpt = n_nodes // NS
    assert n_chunks % nbuf == 0 and n_chunks >= 2 * nbuf
    params = None if d % 128 == 0 else pltpu.CompilerParams(use_tc_tiling_on_sc=False)

    scratch = (
        [pltpu.VMEM((chunk,), jnp.int32) for _ in range(2 * nbuf)]
        + [pltpu.VMEM((chunk, d), _f32) for _ in range(nbuf)]
        + [pltpu.VMEM_SHARED((n_nodes, d), _f32)]
        + [pltpu.SemaphoreType.DMA for _ in range(4 * nbuf)]
    )

    @functools.partial(
        pl.kernel,
        out_type=jax.ShapeDtypeStruct((NC, n_nodes, d), _f32),
        mesh=_sc_mesh(),
        compiler_params=params,
        scratch_types=scratch,
    )
    def seg_sum(table_hbm, src_hbm, dst_hbm, junk_hbm, zeros_hbm, out_hbm, *scr):
        sidx = scr[0:nbuf]
        didx = scr[nbuf:2 * nbuf]
        bufs = scr[2 * nbuf:3 * nbuf]
        acc = scr[3 * nbuf]
        gsem = scr[3 * nbuf + 1:4 * nbuf + 1]
        ssem = scr[4 * nbuf + 1:5 * nbuf + 1]
        isem = scr[5 * nbuf + 1:6 * nbuf + 1]
        dsem = scr[6 * nbuf + 1:7 * nbuf + 1]
        c = lax.axis_index("c")
        s = lax.axis_index("s")
        pltpu.sync_copy(zeros_hbm, acc.at[pl.ds(s * rpt, rpt)])
        plsc.subcore_barrier()

        for q in range(nbuf):
            pltpu.async_copy(src_hbm.at[c, s, q, 0], sidx[q], isem[q])
        for q in range(nbuf - 1):
            pltpu.async_copy(dst_hbm.at[c, s, q, 0], didx[q], dsem[q])
        pltpu.sync_copy(junk_hbm, didx[nbuf - 1])
        pltpu.async_copy(bufs[nbuf - 1], acc.at[didx[nbuf - 1]], ssem[nbuf - 1], add=True)
        for q in range(nbuf - 1):
            pltpu.make_async_copy(src_hbm.at[c, s, q, 0], sidx[q], isem[q]).wait()
            pltpu.async_copy(table_hbm.at[sidx[q]], bufs[q], gsem[q])

        def one(j, b, pre_src, pre_gather):
            pb = (b + nbuf - 1) % nbuf
            pltpu.make_async_copy(table_hbm.at[sidx[b]], bufs[b], gsem[b]).wait()
            if pre_src:
                pltpu.async_copy(src_hbm.at[c, s, j + nbuf, 0], sidx[b], isem[b])
            pltpu.make_async_copy(bufs[pb], acc.at[didx[pb]], ssem[pb]).wait()
            if pre_gather:
                pltpu.async_copy(dst_hbm.at[c, s, j + nbuf - 1, 0], didx[pb], dsem[pb])
                pltpu.make_async_copy(
                    src_hbm.at[c, s, j + nbuf - 1, 0], sidx[pb], isem[pb]
                ).wait()
                pltpu.async_copy(table_hbm.at[sidx[pb]], bufs[pb], gsem[pb])
            pltpu.make_async_copy(dst_hbm.at[c, s, j, 0], didx[b], dsem[b]).wait()
            pltpu.async_copy(bufs[b], acc.at[didx[b]], ssem[b], add=True)

        def steady(i, carry):
            for q in range(nbuf):
                one(i * nbuf + q, q, True, True)
            return carry

        lax.fori_loop(0, (n_chunks - nbuf) // nbuf, steady, 0)
        jt = n_chunks - nbuf
        one(jt, jt % nbuf, False, True)
        for q in range(1, nbuf):
            one(jt + q, (jt + q) % nbuf, False, False)
        pltpu.make_async_copy(
            bufs[(n_chunks - 1) % nbuf],
            acc.at[didx[(n_chunks - 1) % nbuf]],
            ssem[(n_chunks - 1) % nbuf],
        ).wait()

        plsc.subcore_barrier()
        pltpu.sync_copy(
            acc.at[pl.ds(s * rpt, rpt)], out_hbm.at[c, pl.ds(s * rpt, rpt)]
        )

    return seg_sum


def _tc_stage_a(x, w1, degp, bn: int):
    n, d_in = x.shape
    d_h = w1.shape[1]

    def body(x_ref, w1_ref, degp_ref, table_ref, dinv_ref):
        deg = (
            jnp.sum(degp_ref[0], axis=-1) + jnp.sum(degp_ref[1], axis=-1)
        ) * (1.0 / DEGW) + 1.0
        dinv = lax.rsqrt(deg)
        p = jnp.dot(x_ref[...], w1_ref[...], preferred_element_type=_f32)
        table_ref[...] = p * dinv[:, None]
        dinv_ref[...] = dinv[:, None]

    return pl.pallas_call(
        body,
        grid=(n // bn,),
        in_specs=[
            pl.BlockSpec((bn, d_in), lambda i: (i, 0)),
            pl.BlockSpec((d_in, d_h), lambda i: (0, 0)),
            pl.BlockSpec((NC, bn, DEGW), lambda i: (0, i, 0)),
        ],
        out_specs=[
            pl.BlockSpec((bn, d_h), lambda i: (i, 0)),
            pl.BlockSpec((bn, 1), lambda i: (i, 0)),
        ],
        out_shape=[
            jax.ShapeDtypeStruct((n, d_h), _f32),
            jax.ShapeDtypeStruct((n, 1), _f32),
        ],
    )(x, w1, degp)


def _tc_stage_b(aggp, table1, dinv, b1, w2p, bn: int):
    n, d_h = table1.shape
    d2 = w2p.shape[1]

    def body(aggp_ref, t1_ref, dinv_ref, b1_ref, w2_ref, out_ref):
        dv = dinv_ref[...]
        h = dv * (aggp_ref[0] + aggp_ref[1] + t1_ref[...]) + b1_ref[...]
        h = jnp.maximum(h, 0.0)
        q = jnp.dot(h, w2_ref[...], preferred_element_type=_f32)
        out_ref[...] = q * dv

    return pl.pallas_call(
        body,
        grid=(n // bn,),
        in_specs=[
            pl.BlockSpec((NC, bn, d_h), lambda i: (0, i, 0)),
            pl.BlockSpec((bn, d_h), lambda i: (i, 0)),
            pl.BlockSpec((bn, 1), lambda i: (i, 0)),
            pl.BlockSpec((1, d_h), lambda i: (0, 0)),
            pl.BlockSpec((d_h, d2), lambda i: (0, 0)),
        ],
        out_specs=pl.BlockSpec((bn, d2), lambda i: (i, 0)),
        out_shape=jax.ShapeDtypeStruct((n, d2), _f32),
    )(aggp, table1, dinv, b1, w2p)


def _tc_stage_c(aggp, table2, dinv, b2p, d_out: int, bn: int):
    n, d2 = table2.shape

    def body(aggp_ref, t2_ref, dinv_ref, b2_ref, out_ref):
        dv = dinv_ref[...]
        o = dv * (aggp_ref[0] + aggp_ref[1] + t2_ref[...]) + b2_ref[...]
        col = lax.broadcasted_iota(jnp.int32, o.shape, 1)
        valid = col < d_out
        neg = jnp.full_like(o, -jnp.inf)
        logits = jnp.where(valid, o, neg)
        m = jnp.max(logits, axis=-1, keepdims=True)
        lse = jnp.log(jnp.sum(jnp.exp(logits - m), axis=-1, keepdims=True)) + m
        out_ref[...] = (o - lse)[:, :d_out]

    return pl.pallas_call(
        body,
        grid=(n // bn,),
        in_specs=[
            pl.BlockSpec((NC, bn, d2), lambda i: (0, i, 0)),
            pl.BlockSpec((bn, d2), lambda i: (i, 0)),
            pl.BlockSpec((bn, 1), lambda i: (i, 0)),
            pl.BlockSpec((1, d2), lambda i: (0, 0)),
        ],
        out_specs=pl.BlockSpec((bn, d_out), lambda i: (i, 0)),
        out_shape=jax.ShapeDtypeStruct((n, d_out), _f32),
    )(aggp, table2, dinv, b2p)


def kernel(x, edge_index, W1, b1, W2, b2):
    n, d_in = x.shape
    e = edge_index.shape[1]
    d_h = W1.shape[1]
    d_out = W2.shape[1]
    d2 = 48
    npad = -(-n // (NS * 8)) * (NS * 8) + (NS * 8 if n % (NS * 8) == 0 else 0)

    assert e % NW == 0
    epw = e // NW

    def edge_layout(row, n_chunks, chunk, is_dst):
        w = row.reshape(NW, epw)
        npp = n_chunks * chunk - epw
        if npp:
            wid = jnp.arange(NW, dtype=jnp.int32)[:, None]
            k = jnp.arange(npp, dtype=jnp.int32)[None, :]
            pad = (n + (k + wid * 7) % (npad - n)) if is_dst else (k * 131 + wid * 977) % n
            w = jnp.concatenate([w, pad], axis=1)
        return w.reshape(NC, NS, n_chunks, 1, chunk)

    ck1, nb1 = 96, 3
    ck2, nb2 = 128, 4
    nc1 = -(-(-(-epw // ck1)) // nb1) * nb1
    nc2 = -(-(-(-epw // ck2)) // nb2) * nb2
    src1 = edge_layout(edge_index[0], nc1, ck1, False)
    dst1 = edge_layout(edge_index[1], nc1, ck1, True)
    src2 = edge_layout(edge_index[0], nc2, ck2, False)
    dst2 = edge_layout(edge_index[1], nc2, ck2, True)

    rpt = npad // NS
    ones_deg = jnp.ones((CHUNK, DEGW), _f32)
    zeros_deg = jnp.zeros((rpt, DEGW), _f32)
    zeros_h = jnp.zeros((rpt, d_h), _f32)
    zeros_2 = jnp.zeros((rpt, d2), _f32)
    w2p = jnp.pad(W2, ((0, 0), (0, d2 - d_out)))
    b1r = b1.reshape(1, d_h)
    b2p = jnp.pad(b2, (0, d2 - d_out)).reshape(1, d2)

    junk1 = (n + jnp.arange(ck1, dtype=jnp.int32) % (npad - n)).astype(jnp.int32)
    junk2 = (n + jnp.arange(ck2, dtype=jnp.int32) % (npad - n)).astype(jnp.int32)

    degp = _make_deg_kernel(npad, nc2)(dst2, ones_deg, zeros_deg)
    table1, dinv = _tc_stage_a(x, W1, degp, bn=2000)
    aggp1 = _make_seg_sum_kernel(npad, nc1, d_h, ck1, nb1)(
        table1, src1, dst1, junk1, zeros_h)
    table2 = _tc_stage_b(aggp1, table1, dinv, b1r, w2p, bn=2000)
    aggp2 = _make_seg_sum_kernel(npad, nc2, d2, ck2, nb2)(
        table2, src2, dst2, junk2, zeros_2)
    return _tc_stage_c(aggp2, table2, dinv, b2p, d_out, bn=2000)
